# baseline (speedup 1.0000x reference)
import functools
import math

import jax
import jax.numpy as jnp
from jax import lax
from jax.experimental import pallas as pl
from jax.experimental.pallas import tpu as pltpu
from jax.experimental.pallas import tpu_sc as plsc

DIM = 128
SCALE = math.sqrt(float(DIM))

_NC = 2
_NS = 16
_NW = _NC * _NS
_NBUF = 6


@functools.lru_cache(maxsize=None)
def _make_kernel(batch, hist):
    b_per_w = batch // _NW
    assert batch % _NW == 0 and b_per_w % 8 == 0 and b_per_w <= 128
    n_chunks = hist
    assert n_chunks % 2 == 0 and n_chunks >= 6
    mesh = plsc.VectorSubcoreMesh(core_axis_name="c", subcore_axis_name="s")

    @functools.partial(
        pl.kernel,
        mesh=mesh,
        out_type=jax.ShapeDtypeStruct((hist, batch, DIM), jnp.float32),
        scratch_types=[
            pltpu.VMEM((n_chunks, b_per_w), jnp.int32),
            pltpu.VMEM((_NBUF, b_per_w, DIM), jnp.float32),
            [pltpu.SemaphoreType.DMA] * _NBUF,
            [pltpu.SemaphoreType.DMA] * _NBUF,
        ],
        compiler_params=pltpu.CompilerParams(skip_device_barrier=True),
    )
    def body(idx_hbm, table_hbm, out_hbm, idx_v, rows_v, gsem, osem):
        wid = lax.axis_index("s") * _NC + lax.axis_index("c")
        b0 = wid * b_per_w
        pltpu.sync_copy(idx_hbm.at[wid], idx_v)

        def g_copy(j, b):
            return pltpu.make_async_copy(
                table_hbm.at[idx_v.at[j]], rows_v.at[b], gsem[b])

        def o_copy(j, b):
            return pltpu.make_async_copy(
                rows_v.at[b], out_hbm.at[j, pl.ds(b0, b_per_w)], osem[b])

        def scale(b):
            @plsc.parallel_loop(0, b_per_w, step=1, unroll=4)
            def _sb(i):
                for u in range(DIM // 16):
                    sl = pl.ds(u * 16, 16)
                    rows_v[b, i, sl] = rows_v[b, i, sl] * SCALE

        def step(j, b, refill, drain):
            g_copy(j, b).wait()
            scale(b)
            o_copy(j, b).start()
            if refill:
                if drain:
                    o_copy(j - 1, (b - 1) % _NBUF).wait()
                g_copy(j + _NBUF - 1, (b - 1) % _NBUF).start()

        for k in range(_NBUF - 1):
            g_copy(k, k).start()
        step(0, 0, True, False)

        n_steady = n_chunks - _NBUF
        n_main = (n_steady // _NBUF) * _NBUF

        def loop_body(jp, c):
            j = 1 + _NBUF * jp
            for h in range(_NBUF):
                step(j + h, (1 + h) % _NBUF, True, True)
            return c

        lax.fori_loop(0, n_main // _NBUF, loop_body, 0)

        for j in range(1 + n_main, n_steady + 1):
            step(j, j % _NBUF, True, True)
        for j in range(n_steady + 1, n_chunks):
            step(j, j % _NBUF, False, False)
        for j in range(n_chunks - _NBUF, n_chunks):
            o_copy(j, j % _NBUF).wait()

    return body


def kernel(input, table):
    batch, hist = input.shape
    b_per_w = batch // _NW
    idx3 = input.T.reshape(hist, _NW, b_per_w).transpose(1, 0, 2)
    out_t = _make_kernel(batch, hist)(idx3, table)
    return out_t.transpose(1, 0, 2)


# speedup vs baseline: 1.0039x; 1.0039x over previous
import functools
import math

import jax
import jax.numpy as jnp
from jax import lax
from jax.experimental import pallas as pl
from jax.experimental.pallas import tpu as pltpu
from jax.experimental.pallas import tpu_sc as plsc

DIM = 128
SCALE = math.sqrt(float(DIM))

_NC = 2
_NS = 16
_NW = _NC * _NS
_NBUF = 6


@functools.lru_cache(maxsize=None)
def _make_kernel(batch, hist):
    b_per_w = batch // _NW
    assert batch % _NW == 0 and b_per_w % 8 == 0 and b_per_w <= 128
    n_chunks = hist
    assert n_chunks % 2 == 0 and n_chunks >= 6
    mesh = plsc.VectorSubcoreMesh(core_axis_name="c", subcore_axis_name="s")

    @functools.partial(
        pl.kernel,
        mesh=mesh,
        out_type=jax.ShapeDtypeStruct((hist, batch, DIM), jnp.float32),
        scratch_types=[
            pltpu.VMEM((n_chunks, b_per_w), jnp.int32),
            pltpu.VMEM((_NBUF, b_per_w, DIM), jnp.float32),
            [pltpu.SemaphoreType.DMA] * _NBUF,
            [pltpu.SemaphoreType.DMA] * _NBUF,
        ],
    )
    def body(idx_hbm, table_hbm, out_hbm, idx_v, rows_v, gsem, osem):
        wid = lax.axis_index("s") * _NC + lax.axis_index("c")
        b0 = wid * b_per_w
        pltpu.sync_copy(idx_hbm.at[wid], idx_v)

        def g_copy(j, b):
            return pltpu.make_async_copy(
                table_hbm.at[idx_v.at[j]], rows_v.at[b], gsem[b])

        def o_copy(j, b):
            return pltpu.make_async_copy(
                rows_v.at[b], out_hbm.at[j, pl.ds(b0, b_per_w)], osem[b])

        def scale(b):
            @plsc.parallel_loop(0, b_per_w, step=1, unroll=4)
            def _sb(i):
                for u in range(DIM // 16):
                    sl = pl.ds(u * 16, 16)
                    rows_v[b, i, sl] = rows_v[b, i, sl] * SCALE

        def step(j, b, refill, drain):
            g_copy(j, b).wait()
            scale(b)
            o_copy(j, b).start()
            if refill:
                if drain:
                    o_copy(j - 1, (b - 1) % _NBUF).wait()
                g_copy(j + _NBUF - 1, (b - 1) % _NBUF).start()

        for k in range(_NBUF - 1):
            g_copy(k, k).start()
        step(0, 0, True, False)

        n_steady = n_chunks - _NBUF
        n_main = (n_steady // _NBUF) * _NBUF

        def loop_body(jp, c):
            j = 1 + _NBUF * jp
            for h in range(_NBUF):
                step(j + h, (1 + h) % _NBUF, True, True)
            return c

        lax.fori_loop(0, n_main // _NBUF, loop_body, 0)

        for j in range(1 + n_main, n_steady + 1):
            step(j, j % _NBUF, True, True)
        for j in range(n_steady + 1, n_chunks):
            step(j, j % _NBUF, False, False)
        for j in range(n_chunks - _NBUF, n_chunks):
            o_copy(j, j % _NBUF).wait()

    return body


def kernel(input, table):
    batch, hist = input.shape
    b_per_w = batch // _NW
    idx3 = input.T.reshape(hist, _NW, b_per_w).transpose(1, 0, 2)
    out_t = _make_kernel(batch, hist)(idx3, table)
    return out_t.transpose(1, 0, 2)
